# baseline (speedup 1.0000x reference)
import functools
import math

import jax
import jax.numpy as jnp
from jax import lax
from jax.experimental import pallas as pl
from jax.experimental.pallas import tpu as pltpu
from jax.experimental.pallas import tpu_sc as plsc

N = 100008
D = 192
H = 8
DH = D // H
SET = 36
NL = 8
FF = 384

DW = 256
GRP = 2 * SET
RB = 576
NP = 100224
NSC = 100096

_NW = 32
_CH = 128

_PREC = lax.Precision.DEFAULT


def _sc_mesh():
    return plsc.VectorSubcoreMesh(core_axis_name="c", subcore_axis_name="s")


_SC_UNTILED = pltpu.CompilerParams(use_tc_tiling_on_sc=False)


def _wid():
    return lax.axis_index("s") * 2 + lax.axis_index("c")


def _make_gather(nt, w, dtype, b_out, tiled):
    nchunks = (b_out + _CH - 1) // _CH
    tail = b_out - (nchunks - 1) * _CH
    kmax = (nchunks + _NW - 1) // _NW
    if w == 1:
        row_shape, out_shape = (_CH,), (b_out,)
    else:
        row_shape, out_shape = (_CH, w), (b_out, w)

    @functools.partial(
        pl.kernel,
        out_type=jax.ShapeDtypeStruct(out_shape, dtype),
        mesh=_sc_mesh(),
        scratch_types=[
            pltpu.VMEM((_CH,), jnp.int32),
            pltpu.VMEM((_CH,), jnp.int32),
            pltpu.VMEM(row_shape, dtype),
            pltpu.VMEM(row_shape, dtype),
            pltpu.SemaphoreType.DMA,
            pltpu.SemaphoreType.DMA,
        ],
        compiler_params=None if tiled else _SC_UNTILED,
    )
    def gk(table, idxs, out, i0, i1, r0, r1, s0, s1):
        wid = _wid()
        ibufs, rbufs, sems = (i0, i1), (r0, r1), (s0, s1)

        def issue(k):
            ci = wid + k * _NW
            b = k % 2

            @pl.when(ci < nchunks)
            def _():
                pltpu.sync_copy(idxs.at[pl.ds(ci * _CH, _CH)], ibufs[b])
                pltpu.async_copy(table.at[ibufs[b]], rbufs[b], sems[b])

        def drain(k):
            ci = wid + k * _NW
            b = k % 2

            @pl.when(ci < nchunks)
            def _():
                pltpu.make_async_copy(table.at[ibufs[b]], rbufs[b],
                                      sems[b]).wait()
                if tail == _CH:
                    pltpu.sync_copy(rbufs[b], out.at[pl.ds(ci * _CH, _CH)])
                else:
                    @pl.when(ci < nchunks - 1)
                    def _():
                        pltpu.sync_copy(rbufs[b],
                                        out.at[pl.ds(ci * _CH, _CH)])

                    @pl.when(ci == nchunks - 1)
                    def _():
                        pltpu.sync_copy(rbufs[b].at[pl.ds(0, tail)],
                                        out.at[pl.ds(ci * _CH, tail)])

        issue(0)
        for k in range(kmax):
            if k + 1 < kmax:
                issue(k + 1)
            drain(k)

    return gk


def _make_scatter(b):
    nchunks = b // _CH
    kmax = (nchunks + _NW - 1) // _NW

    @functools.partial(
        pl.kernel,
        out_type=jax.ShapeDtypeStruct((b,), jnp.int32),
        mesh=_sc_mesh(),
        scratch_types=[
            pltpu.VMEM((_CH,), jnp.int32),
            pltpu.VMEM((_CH,), jnp.int32),
            pltpu.VMEM((_CH,), jnp.int32),
            pltpu.VMEM((_CH,), jnp.int32),
            pltpu.SemaphoreType.DMA,
            pltpu.SemaphoreType.DMA,
        ],
        compiler_params=_SC_UNTILED,
    )
    def sk(vals, idxs, out, i0, i1, v0, v1, s0, s1):
        wid = _wid()
        ibufs, vbufs, sems = (i0, i1), (v0, v1), (s0, s1)

        def issue(k):
            ci = wid + k * _NW
            b = k % 2

            @pl.when(ci < nchunks)
            def _():
                pltpu.sync_copy(idxs.at[pl.ds(ci * _CH, _CH)], ibufs[b])
                pltpu.sync_copy(vals.at[pl.ds(ci * _CH, _CH)], vbufs[b])
                pltpu.async_copy(vbufs[b], out.at[ibufs[b]], sems[b])

        def drain(k):
            ci = wid + k * _NW
            b = k % 2

            @pl.when(ci < nchunks)
            def _():
                pltpu.make_async_copy(vbufs[b], out.at[ibufs[b]],
                                      sems[b]).wait()

        issue(0)
        for k in range(kmax):
            if k + 1 < kmax:
                issue(k + 1)
            drain(k)

    return sk


def _dot(a, b):
    return lax.dot_general(a, b, (((a.ndim - 1,), (0,)), ((), ())),
                           preferred_element_type=jnp.float32,
                           precision=_PREC)


def _dot_nt(a, b):
    return lax.dot_general(a, b, (((1,), (1,)), ((), ())),
                           preferred_element_type=jnp.float32,
                           precision=_PREC)


def _ln(x, g, b):
    m = jnp.mean(x, axis=-1, keepdims=True)
    xc = x - m
    v = jnp.mean(xc * xc, axis=-1, keepdims=True)
    return xc * lax.rsqrt(v + 1e-5) * g + b


def _gelu(x):
    c = math.sqrt(2.0 / math.pi)
    return 0.5 * x * (1.0 + jnp.tanh(c * (x + 0.044715 * x * x * x)))


def _layer_body(x_ref, hm_ref, bias_ref,
                wp1_ref, bp1_ref, wp2_ref, bp2_ref,
                wq_ref, bq_ref, wk_ref, bk_ref, wv_ref, bv_ref,
                wo_ref, bo_ref, g1_ref, be1_ref,
                w1_ref, b1_ref, w2_ref, b2_ref, g2_ref, be2_ref, y_ref):
    x = x_ref[:, :D]
    cy = x_ref[:, D:D + 1]
    cx = x_ref[:, D + 1:D + 2]

    h = jnp.maximum(cy * wp1_ref[0:1, :] + cx * wp1_ref[1:2, :]
                    + bp1_ref[...], 0.0)
    pos = _dot(h, wp2_ref[...]) + bp2_ref[...]

    qk_in = x + pos
    qp = _dot(qk_in, wq_ref[...]) + bq_ref[...]
    kp = _dot(qk_in, wk_ref[...]) + bk_ref[...]
    vp = _dot(x, wv_ref[...]) + bv_ref[...]

    headmask = hm_ref[...]
    bias = bias_ref[...]

    outs = []
    zkv = jnp.zeros((128 - GRP, D), jnp.float32)
    for g in range(RB // GRP):
        q_g = qp[g * GRP:(g + 1) * GRP, :]
        k_g = jnp.concatenate([kp[g * GRP:(g + 1) * GRP, :], zkv], axis=0)
        v_g = jnp.concatenate([vp[g * GRP:(g + 1) * GRP, :], zkv], axis=0)
        qs = jnp.concatenate([q_g] * H, axis=0) * headmask
        s_g = _dot_nt(qs, k_g) + bias
        m = jnp.max(s_g, axis=1, keepdims=True)
        e = jnp.exp(s_g - m)
        r = 1.0 / jnp.sum(e, axis=1, keepdims=True)
        o = _dot(e * r, v_g) * headmask
        acc = o[0:GRP, :]
        for hh in range(1, H):
            acc = acc + o[hh * GRP:(hh + 1) * GRP, :]
        outs.append(acc)
    att = jnp.concatenate(outs, axis=0)

    att = _dot(att, wo_ref[...]) + bo_ref[...]
    y1 = _ln(x + att, g1_ref[...], be1_ref[...])
    f = _gelu(_dot(y1, w1_ref[...]) + b1_ref[...])
    f2 = _dot(f, w2_ref[...]) + b2_ref[...]
    y = _ln(y1 + f2, g2_ref[...], be2_ref[...])
    y_ref[...] = jnp.concatenate(
        [y, x_ref[:, D:D + 2],
         jnp.zeros((RB, DW - D - 2), jnp.float32)], axis=1)


def _tc_layer(xp, masks, wts):
    full = lambda a: pl.BlockSpec(a.shape, lambda i: tuple(0 for _ in a.shape))
    in_specs = [
        pl.BlockSpec((RB, DW), lambda i: (i, 0)),
    ] + [full(m) for m in masks] + [full(w) for w in wts]
    return pl.pallas_call(
        _layer_body,
        grid=(NP // RB,),
        in_specs=in_specs,
        out_specs=pl.BlockSpec((RB, DW), lambda i: (i, 0)),
        out_shape=jax.ShapeDtypeStruct((NP, DW), jnp.float32),
        compiler_params=pltpu.CompilerParams(
            dimension_semantics=("arbitrary",)),
    )(xp, *masks, *wts)


_CB = 216


def _pad_lanes(x, c):
    def body(x_ref, c_ref, o_ref):
        o_ref[...] = jnp.concatenate(
            [x_ref[...], c_ref[...],
             jnp.zeros((_CB, DW - D - 2), jnp.float32)], axis=1)
    return pl.pallas_call(
        body,
        grid=(N // _CB,),
        in_specs=[pl.BlockSpec((_CB, D), lambda i: (i, 0)),
                  pl.BlockSpec((_CB, 2), lambda i: (i, 0))],
        out_specs=pl.BlockSpec((_CB, DW), lambda i: (i, 0)),
        out_shape=jax.ShapeDtypeStruct((N, DW), jnp.float32),
    )(x, c)


def _drop_lanes(x):
    def body(x_ref, o_ref):
        o_ref[...] = x_ref[:, :D]
    return pl.pallas_call(
        body,
        grid=(N // _CB,),
        in_specs=[pl.BlockSpec((_CB, DW), lambda i: (i, 0))],
        out_specs=pl.BlockSpec((_CB, D), lambda i: (i, 0)),
        out_shape=jax.ShapeDtypeStruct((N, D), jnp.float32),
    )(x)


def kernel(voxel_features, voxel_coords, set_inds, set_mask, coors_in_win,
           Wqkv, bqkv, Wo, bo, W1, b1, W2, b2, g1, be1, g2, be2,
           Wp1, bp1, Wp2, bp2):
    P0 = set_inds[0].reshape(N).astype(jnp.int32)
    P1 = set_inds[1].reshape(N).astype(jnp.int32)

    tail_sc = jnp.arange(N, NSC, dtype=jnp.int32)
    vals = jnp.arange(NSC, dtype=jnp.int32)
    scat = _make_scatter(NSC)
    invP0 = scat(vals, jnp.concatenate([P0, tail_sc]))
    invP1 = scat(vals, jnp.concatenate([P1, tail_sc]))

    zpad = jnp.zeros((NP - N,), jnp.int32)
    P0p = jnp.concatenate([P0, zpad])
    P1p = jnp.concatenate([P1, zpad])

    g1d = _make_gather(NSC, 1, jnp.int32, NP, tiled=False)
    A = [g1d(invP0, P1p), g1d(invP1, P0p)]

    w8 = H * GRP
    rr = jnp.arange(w8, dtype=jnp.int32)[:, None]
    cc = jnp.arange(D, dtype=jnp.int32)[None, :]
    headmask = (rr // GRP == cc // DH).astype(jnp.float32)
    c128 = jnp.arange(128, dtype=jnp.int32)[None, :]
    bias = jnp.where((c128 // SET) == ((rr % GRP) // SET), 0.0, -1e9)
    masks = (headmask, bias)

    gfeat0 = _make_gather(N, DW, jnp.float32, NP, tiled=True)
    gfeat = _make_gather(NP, DW, jnp.float32, NP, tiled=True)
    gfin = _make_gather(NP, DW, jnp.float32, N, tiled=True)

    xp = gfeat0(_pad_lanes(voxel_features, coors_in_win), P0p)
    y = xp
    scale = 1.0 / math.sqrt(DH)
    for i in range(NL):
        s = i % 2
        row = lambda a: a[i].reshape(1, -1)
        wts = (Wp1[i], row(bp1), Wp2[i], row(bp2),
               Wqkv[i, :, :D] * scale, bqkv[i, :D].reshape(1, D) * scale,
               Wqkv[i, :, D:2 * D], bqkv[i, D:2 * D].reshape(1, D),
               Wqkv[i, :, 2 * D:], bqkv[i, 2 * D:].reshape(1, D),
               Wo[i], row(bo), row(g1), row(be1),
               W1[i], row(b1), W2[i], row(b2), row(g2), row(be2))
        y = _tc_layer(xp, masks, wts)
        if i + 1 < NL:
            xp = gfeat(y, A[s])

    return _drop_lanes(gfin(y, invP1))


# speedup vs baseline: 1.2175x; 1.2175x over previous
import functools
import math

import jax
import jax.numpy as jnp
from jax import lax
from jax.experimental import pallas as pl
from jax.experimental.pallas import tpu as pltpu
from jax.experimental.pallas import tpu_sc as plsc

N = 100008
D = 192
H = 8
DH = D // H
SET = 36
NL = 8
FF = 384

DW = 256
GRP = 2 * SET
RB = 3456
NP = 100224
NSC = 100096

_NW = 32
_CH = 128

_PREC = lax.Precision.DEFAULT


def _sc_mesh():
    return plsc.VectorSubcoreMesh(core_axis_name="c", subcore_axis_name="s")


_SC_UNTILED = pltpu.CompilerParams(use_tc_tiling_on_sc=False)


def _wid():
    return lax.axis_index("s") * 2 + lax.axis_index("c")


def _make_gather(nt, w, dtype, b_out, tiled):
    nchunks = (b_out + _CH - 1) // _CH
    tail = b_out - (nchunks - 1) * _CH
    kmax = (nchunks + _NW - 1) // _NW
    if w == 1:
        row_shape, out_shape = (_CH,), (b_out,)
    else:
        row_shape, out_shape = (_CH, w), (b_out, w)

    @functools.partial(
        pl.kernel,
        out_type=jax.ShapeDtypeStruct(out_shape, dtype),
        mesh=_sc_mesh(),
        scratch_types=[
            pltpu.VMEM((_CH,), jnp.int32),
            pltpu.VMEM((_CH,), jnp.int32),
            pltpu.VMEM(row_shape, dtype),
            pltpu.VMEM(row_shape, dtype),
            pltpu.SemaphoreType.DMA,
            pltpu.SemaphoreType.DMA,
        ],
        compiler_params=None if tiled else _SC_UNTILED,
    )
    def gk(table, idxs, out, i0, i1, r0, r1, s0, s1):
        wid = _wid()
        ibufs, rbufs, sems = (i0, i1), (r0, r1), (s0, s1)

        def issue(k):
            ci = wid + k * _NW
            b = k % 2

            @pl.when(ci < nchunks)
            def _():
                pltpu.sync_copy(idxs.at[pl.ds(ci * _CH, _CH)], ibufs[b])
                pltpu.async_copy(table.at[ibufs[b]], rbufs[b], sems[b])

        def drain(k):
            ci = wid + k * _NW
            b = k % 2

            @pl.when(ci < nchunks)
            def _():
                pltpu.make_async_copy(table.at[ibufs[b]], rbufs[b],
                                      sems[b]).wait()
                if tail == _CH:
                    pltpu.sync_copy(rbufs[b], out.at[pl.ds(ci * _CH, _CH)])
                else:
                    @pl.when(ci < nchunks - 1)
                    def _():
                        pltpu.sync_copy(rbufs[b],
                                        out.at[pl.ds(ci * _CH, _CH)])

                    @pl.when(ci == nchunks - 1)
                    def _():
                        pltpu.sync_copy(rbufs[b].at[pl.ds(0, tail)],
                                        out.at[pl.ds(ci * _CH, tail)])

        issue(0)
        for k in range(kmax):
            if k + 1 < kmax:
                issue(k + 1)
            drain(k)

    return gk


def _make_scatter(b):
    nchunks = b // _CH
    kmax = (nchunks + _NW - 1) // _NW

    @functools.partial(
        pl.kernel,
        out_type=jax.ShapeDtypeStruct((b,), jnp.int32),
        mesh=_sc_mesh(),
        scratch_types=[
            pltpu.VMEM((_CH,), jnp.int32),
            pltpu.VMEM((_CH,), jnp.int32),
            pltpu.VMEM((_CH,), jnp.int32),
            pltpu.VMEM((_CH,), jnp.int32),
            pltpu.SemaphoreType.DMA,
            pltpu.SemaphoreType.DMA,
        ],
        compiler_params=_SC_UNTILED,
    )
    def sk(vals, idxs, out, i0, i1, v0, v1, s0, s1):
        wid = _wid()
        ibufs, vbufs, sems = (i0, i1), (v0, v1), (s0, s1)

        def issue(k):
            ci = wid + k * _NW
            b = k % 2

            @pl.when(ci < nchunks)
            def _():
                pltpu.sync_copy(idxs.at[pl.ds(ci * _CH, _CH)], ibufs[b])
                pltpu.sync_copy(vals.at[pl.ds(ci * _CH, _CH)], vbufs[b])
                pltpu.async_copy(vbufs[b], out.at[ibufs[b]], sems[b])

        def drain(k):
            ci = wid + k * _NW
            b = k % 2

            @pl.when(ci < nchunks)
            def _():
                pltpu.make_async_copy(vbufs[b], out.at[ibufs[b]],
                                      sems[b]).wait()

        issue(0)
        for k in range(kmax):
            if k + 1 < kmax:
                issue(k + 1)
            drain(k)

    return sk


def _dot(a, b):
    return lax.dot_general(a, b, (((a.ndim - 1,), (0,)), ((), ())),
                           preferred_element_type=jnp.float32,
                           precision=_PREC)


def _dot_nt(a, b):
    return lax.dot_general(a, b, (((1,), (1,)), ((), ())),
                           preferred_element_type=jnp.float32,
                           precision=_PREC)


def _ln(x, g, b):
    m = jnp.mean(x, axis=-1, keepdims=True)
    xc = x - m
    v = jnp.mean(xc * xc, axis=-1, keepdims=True)
    return xc * lax.rsqrt(v + 1e-5) * g + b


def _gelu(x):
    c = math.sqrt(2.0 / math.pi)
    return 0.5 * x * (1.0 + jnp.tanh(c * (x + 0.044715 * x * x * x)))


def _layer_body(x_ref, hm_ref, bias_ref,
                wp1_ref, bp1_ref, wp2_ref, bp2_ref,
                wq_ref, bq_ref, wk_ref, bk_ref, wv_ref, bv_ref,
                wo_ref, bo_ref, g1_ref, be1_ref,
                w1_ref, b1_ref, w2_ref, b2_ref, g2_ref, be2_ref, y_ref):
    x = x_ref[:, :D]
    cy = x_ref[:, D:D + 1]
    cx = x_ref[:, D + 1:D + 2]

    h = jnp.maximum(cy * wp1_ref[0:1, :] + cx * wp1_ref[1:2, :]
                    + bp1_ref[...], 0.0)
    pos = _dot(h, wp2_ref[...]) + bp2_ref[...]

    qk_in = x + pos
    qp = _dot(qk_in, wq_ref[...]) + bq_ref[...]
    kp = _dot(qk_in, wk_ref[...]) + bk_ref[...]
    vp = _dot(x, wv_ref[...]) + bv_ref[...]

    headmask = hm_ref[...]
    bias = bias_ref[...]

    outs = []
    zkv = jnp.zeros((128 - GRP, D), jnp.float32)
    for g in range(RB // GRP):
        q_g = qp[g * GRP:(g + 1) * GRP, :]
        k_g = jnp.concatenate([kp[g * GRP:(g + 1) * GRP, :], zkv], axis=0)
        v_g = jnp.concatenate([vp[g * GRP:(g + 1) * GRP, :], zkv], axis=0)
        qs = jnp.concatenate([q_g] * H, axis=0) * headmask
        s_g = _dot_nt(qs, k_g) + bias
        e = jnp.exp(s_g)
        r = 1.0 / jnp.sum(e, axis=1, keepdims=True)
        o = _dot(e * r, v_g) * headmask
        acc = o[0:GRP, :]
        for hh in range(1, H):
            acc = acc + o[hh * GRP:(hh + 1) * GRP, :]
        outs.append(acc)
    att = jnp.concatenate(outs, axis=0)

    att = _dot(att, wo_ref[...]) + bo_ref[...]
    y1 = _ln(x + att, g1_ref[...], be1_ref[...])
    f = _gelu(_dot(y1, w1_ref[...]) + b1_ref[...])
    f2 = _dot(f, w2_ref[...]) + b2_ref[...]
    y = _ln(y1 + f2, g2_ref[...], be2_ref[...])
    y_ref[...] = jnp.concatenate(
        [y, x_ref[:, D:D + 2],
         jnp.zeros((RB, DW - D - 2), jnp.float32)], axis=1)


def _tc_layer(xp, masks, wts):
    full = lambda a: pl.BlockSpec(a.shape, lambda i: tuple(0 for _ in a.shape))
    in_specs = [
        pl.BlockSpec((RB, DW), lambda i: (i, 0)),
    ] + [full(m) for m in masks] + [full(w) for w in wts]
    return pl.pallas_call(
        _layer_body,
        grid=(NP // RB,),
        in_specs=in_specs,
        out_specs=pl.BlockSpec((RB, DW), lambda i: (i, 0)),
        out_shape=jax.ShapeDtypeStruct((NP, DW), jnp.float32),
        compiler_params=pltpu.CompilerParams(
            dimension_semantics=("arbitrary",)),
    )(xp, *masks, *wts)


_CB = 216


def _pad_lanes(x, c):
    def body(x_ref, c_ref, o_ref):
        o_ref[...] = jnp.concatenate(
            [x_ref[...], c_ref[...],
             jnp.zeros((_CB, DW - D - 2), jnp.float32)], axis=1)
    return pl.pallas_call(
        body,
        grid=(N // _CB,),
        in_specs=[pl.BlockSpec((_CB, D), lambda i: (i, 0)),
                  pl.BlockSpec((_CB, 2), lambda i: (i, 0))],
        out_specs=pl.BlockSpec((_CB, DW), lambda i: (i, 0)),
        out_shape=jax.ShapeDtypeStruct((N, DW), jnp.float32),
    )(x, c)


def _drop_lanes(x):
    def body(x_ref, o_ref):
        o_ref[...] = x_ref[:, :D]
    return pl.pallas_call(
        body,
        grid=(N // _CB,),
        in_specs=[pl.BlockSpec((_CB, DW), lambda i: (i, 0))],
        out_specs=pl.BlockSpec((_CB, D), lambda i: (i, 0)),
        out_shape=jax.ShapeDtypeStruct((N, D), jnp.float32),
    )(x)


def kernel(voxel_features, voxel_coords, set_inds, set_mask, coors_in_win,
           Wqkv, bqkv, Wo, bo, W1, b1, W2, b2, g1, be1, g2, be2,
           Wp1, bp1, Wp2, bp2):
    P0 = set_inds[0].reshape(N).astype(jnp.int32)
    P1 = set_inds[1].reshape(N).astype(jnp.int32)

    tail_sc = jnp.arange(N, NSC, dtype=jnp.int32)
    vals = jnp.arange(NSC, dtype=jnp.int32)
    scat = _make_scatter(NSC)
    invP0 = scat(vals, jnp.concatenate([P0, tail_sc]))
    invP1 = scat(vals, jnp.concatenate([P1, tail_sc]))

    zpad = jnp.zeros((NP - N,), jnp.int32)
    P0p = jnp.concatenate([P0, zpad])
    P1p = jnp.concatenate([P1, zpad])

    g1d = _make_gather(NSC, 1, jnp.int32, NP, tiled=False)
    A = [g1d(invP0, P1p), g1d(invP1, P0p)]

    w8 = H * GRP
    rr = jnp.arange(w8, dtype=jnp.int32)[:, None]
    cc = jnp.arange(D, dtype=jnp.int32)[None, :]
    headmask = (rr // GRP == cc // DH).astype(jnp.float32)
    c128 = jnp.arange(128, dtype=jnp.int32)[None, :]
    bias = jnp.where((c128 // SET) == ((rr % GRP) // SET), 0.0, -1e9)
    masks = (headmask, bias)

    gfeat0 = _make_gather(N, DW, jnp.float32, NP, tiled=True)
    gfeat = _make_gather(NP, DW, jnp.float32, NP, tiled=True)
    gfin = _make_gather(NP, DW, jnp.float32, N, tiled=True)

    xp = gfeat0(_pad_lanes(voxel_features, coors_in_win), P0p)
    y = xp
    scale = 1.0 / math.sqrt(DH)
    for i in range(NL):
        s = i % 2
        row = lambda a: a[i].reshape(1, -1)
        wts = (Wp1[i], row(bp1), Wp2[i], row(bp2),
               Wqkv[i, :, :D] * scale, bqkv[i, :D].reshape(1, D) * scale,
               Wqkv[i, :, D:2 * D], bqkv[i, D:2 * D].reshape(1, D),
               Wqkv[i, :, 2 * D:], bqkv[i, 2 * D:].reshape(1, D),
               Wo[i], row(bo), row(g1), row(be1),
               W1[i], row(b1), W2[i], row(b2), row(g2), row(be2))
        y = _tc_layer(xp, masks, wts)
        if i + 1 < NL:
            xp = gfeat(y, A[s])

    return _drop_lanes(gfin(y, invP1))
